# baseline (speedup 1.0000x reference)
import jax
import jax.numpy as jnp
from jax import lax
from jax.experimental import pallas as pl
from jax.experimental.pallas import tpu as pltpu
from jax.experimental.pallas import tpu_sc as plsc

_NUM_BUCKETS = 1000
_EMBED_DIM = 32
_BATCH = 16384
_TOK_LEN = 20
_NORM_MEAN = 0.5
_NORM_STD = 0.2887
_A_D = 69
_B_D = 32

_NC = 2
_NS = 16
_NW = _NC * _NS
_ROWS_PER_W = _BATCH // _NW
_CHUNK = 128
_NCHUNK = _ROWS_PER_W // _CHUNK
_NGROUP = _CHUNK // 16

_ZERO_ROW = 1002


def _body_a(t_hbm, rate_hbm, occl_hbm, age_hbm, gen_hbm, tok_hbm,
            tstab_hbm, occtab_hbm, bnd_hbm, out_hbm,
            occ_v, ts_v, bnd_v, t_v, rate_v, occl_v, age_v, gen_v, tok_v,
            out_v):
    wid = lax.axis_index("s") * _NC + lax.axis_index("c")
    base0 = wid * _ROWS_PER_W

    pltpu.sync_copy(occtab_hbm, occ_v.at[pl.ds(0, (_NUM_BUCKETS + 2) * _EMBED_DIM)])
    pltpu.sync_copy(tstab_hbm, ts_v)
    pltpu.sync_copy(bnd_hbm, bnd_v)
    pltpu.sync_copy(t_hbm.at[pl.ds(base0, _ROWS_PER_W)], t_v)
    pltpu.sync_copy(rate_hbm.at[pl.ds(base0, _ROWS_PER_W)], rate_v)
    pltpu.sync_copy(occl_hbm.at[pl.ds(base0, _ROWS_PER_W)], occl_v)
    pltpu.sync_copy(age_hbm.at[pl.ds(base0, _ROWS_PER_W)], age_v)
    pltpu.sync_copy(gen_hbm.at[pl.ds(base0, _ROWS_PER_W)], gen_v)
    pltpu.sync_copy(tok_hbm.at[pl.ds(base0 * _TOK_LEN, _ROWS_PER_W * _TOK_LEN)],
                    tok_v)
    zeros16 = jnp.zeros((16,), jnp.float32)
    occ_v[pl.ds(_ZERO_ROW * _EMBED_DIM, 16)] = zeros16
    occ_v[pl.ds(_ZERO_ROW * _EMBED_DIM + 16, 16)] = zeros16

    iot = lax.iota(jnp.int32, 16)
    iot69 = iot * _A_D
    iot20 = iot * _TOK_LEN

    def chunk_body(ci, carry):
        @plsc.parallel_loop(0, _NGROUP)
        def group_body(g):
            r0 = g * 16
            w0 = ci * _CHUNK + r0
            fi = r0 * _A_D + iot69

            t = t_v[pl.ds(w0, 16)]
            k0 = jnp.clip((t * float(_NUM_BUCKETS - 1)).astype(jnp.int32) + 1,
                          1, _NUM_BUCKETS)
            b_lo = plsc.load_gather(bnd_v, [k0 - 1])
            b_hi = plsc.load_gather(bnd_v, [k0])
            idx = (k0 - (t < b_lo).astype(jnp.int32)
                   + (t >= b_hi).astype(jnp.int32))
            idx32 = jnp.clip(idx, 0, _NUM_BUCKETS + 1) * _EMBED_DIM

            nt = (t - _NORM_MEAN) / _NORM_STD
            rate = rate_v[pl.ds(w0, 16)]
            occl = occl_v[pl.ds(w0, 16)].astype(jnp.float32)
            age = age_v[pl.ds(w0, 16)]
            gen = gen_v[pl.ds(w0, 16)].astype(jnp.float32)
            plsc.store_scatter(out_v, [fi + 32], nt)
            plsc.store_scatter(out_v, [fi + 33], rate)
            plsc.store_scatter(out_v, [fi + 34], occl)
            plsc.store_scatter(out_v, [fi + 35], age)
            plsc.store_scatter(out_v, [fi + 36], gen)

            tokbase = w0 * _TOK_LEN + iot20
            tok32 = []
            n0 = jnp.zeros((16,), jnp.int32)
            for l in range(_TOK_LEN):
                tk = plsc.load_gather(tok_v, [tokbase + l])
                z = tk == 0
                n0 = n0 + z.astype(jnp.int32)
                tok32.append(jnp.where(z, _ZERO_ROW, tk) * _EMBED_DIM)
            cnt = jnp.maximum(jnp.float32(_TOK_LEN) - n0.astype(jnp.float32), 1.0)
            inv = 1.0 / cnt

            for d in range(_EMBED_DIM):
                tvec = plsc.load_gather(ts_v, [idx32 + d])
                plsc.store_scatter(out_v, [fi + d], tvec)
                accs = [plsc.load_gather(occ_v, [tok32[l] + d])
                        for l in range(4)]
                for l in range(4, _TOK_LEN):
                    accs[l % 4] = accs[l % 4] + plsc.load_gather(
                        occ_v, [tok32[l] + d])
                acc = (accs[0] + accs[1]) + (accs[2] + accs[3])
                plsc.store_scatter(out_v, [fi + (37 + d)], acc * inv)

        pltpu.sync_copy(out_v,
                        out_hbm.at[pl.ds((base0 + ci * _CHUNK) * _A_D,
                                         _CHUNK * _A_D)])
        return carry

    lax.fori_loop(0, _NCHUNK, chunk_body, 0)


def _body_b(uid_hbm, utab_hbm, out_hbm,
            uid_v, blk_v, out_v, sem):
    wid = lax.axis_index("s") * _NC + lax.axis_index("c")
    base0 = wid * _ROWS_PER_W

    pltpu.sync_copy(uid_hbm.at[pl.ds(base0, _ROWS_PER_W)], uid_v)

    iot = lax.iota(jnp.int32, 16)
    iot32 = iot * _B_D

    def chunk_body(ci, carry):
        def group_body(g, c2):
            r0 = g * 16
            w0 = ci * _CHUNK + r0
            fi = r0 * _B_D + iot32
            us = uid_v[pl.ds(w0, 16)]
            for j in range(16):
                u = us[j]
                c0 = pl.multiple_of((u >> 7) * 128, 128)
                pltpu.async_copy(utab_hbm.at[:, pl.ds(c0, 128)],
                                 blk_v.at[j], sem)
            for j in range(16):
                pltpu.make_async_copy(utab_hbm.at[:, pl.ds(0, 128)],
                                      blk_v.at[j], sem).wait()
            col = uid_v[pl.ds(w0, 16)] & 127
            for d in range(_EMBED_DIM):
                uvec = plsc.load_gather(
                    blk_v, [iot, jnp.full((16,), d, jnp.int32), col])
                plsc.store_scatter(out_v, [fi + d], uvec)
            return c2

        lax.fori_loop(0, _NGROUP, group_body, 0)
        pltpu.sync_copy(out_v,
                        out_hbm.at[pl.ds((base0 + ci * _CHUNK) * _B_D,
                                         _CHUNK * _B_D)])
        return carry

    lax.fori_loop(0, _NCHUNK, chunk_body, 0)


_mesh = plsc.VectorSubcoreMesh(core_axis_name="c", subcore_axis_name="s",
                               num_cores=_NC, num_subcores=_NS)
_params = pltpu.CompilerParams(needs_layout_passes=False,
                               use_tc_tiling_on_sc=False)

_call_a = pl.kernel(
    _body_a,
    out_type=jax.ShapeDtypeStruct((_BATCH * _A_D,), jnp.float32),
    mesh=_mesh,
    scratch_types=[
        pltpu.VMEM(((_NUM_BUCKETS + 3) * _EMBED_DIM,), jnp.float32),
        pltpu.VMEM(((_NUM_BUCKETS + 2) * _EMBED_DIM,), jnp.float32),
        pltpu.VMEM((_NUM_BUCKETS + 8,), jnp.float32),
        pltpu.VMEM((_ROWS_PER_W,), jnp.float32),
        pltpu.VMEM((_ROWS_PER_W,), jnp.float32),
        pltpu.VMEM((_ROWS_PER_W,), jnp.int32),
        pltpu.VMEM((_ROWS_PER_W,), jnp.float32),
        pltpu.VMEM((_ROWS_PER_W,), jnp.int32),
        pltpu.VMEM((_ROWS_PER_W * _TOK_LEN,), jnp.int32),
        pltpu.VMEM((_CHUNK * _A_D,), jnp.float32),
    ],
    compiler_params=_params,
)

_call_b = pl.kernel(
    _body_b,
    out_type=jax.ShapeDtypeStruct((_BATCH * _B_D,), jnp.float32),
    mesh=_mesh,
    scratch_types=[
        pltpu.VMEM((_ROWS_PER_W,), jnp.int32),
        pltpu.VMEM((16, _EMBED_DIM, 128), jnp.float32),
        pltpu.VMEM((_CHUNK * _B_D,), jnp.float32),
        pltpu.SemaphoreType.DMA,
    ],
    compiler_params=pltpu.CompilerParams(needs_layout_passes=False,
                                         use_tc_tiling_on_sc=True),
)


@jax.jit
def kernel(user_id, timestamp, user_rating, user_occupation_label,
           raw_user_age, user_gender, occ_tokens, user_table, ts_table,
           occ_table):
    boundaries = jnp.linspace(0.0, 1.0, _NUM_BUCKETS).astype(jnp.float32)
    bnd = jnp.concatenate([boundaries, jnp.full((8,), 2.0, jnp.float32)])
    out_a = _call_a(timestamp, user_rating, user_occupation_label,
                    raw_user_age, user_gender, occ_tokens.reshape(-1),
                    ts_table.reshape(-1), occ_table.reshape(-1), bnd)
    out_b = _call_b(user_id, user_table.T)
    return jnp.concatenate([out_b.reshape(_BATCH, _B_D),
                            out_a.reshape(_BATCH, _A_D)], axis=1)


# speedup vs baseline: 1.3605x; 1.3605x over previous
import jax
import jax.numpy as jnp
from jax import lax
from jax.experimental import pallas as pl
from jax.experimental.pallas import tpu as pltpu
from jax.experimental.pallas import tpu_sc as plsc

_NUM_BUCKETS = 1000
_EMBED_DIM = 32
_BATCH = 16384
_TOK_LEN = 20
_NORM_MEAN = 0.5
_NORM_STD = 0.2887
_OUT_D = 101

_NC = 2
_NS = 16
_NW = _NC * _NS
_ROWS_PER_W = _BATCH // _NW
_CHUNK = 128
_NCHUNK = _ROWS_PER_W // _CHUNK
_NGROUP = _CHUNK // 16

_ZERO_ROW = 1002


def _body(uid_hbm, t_hbm, rate_hbm, occl_hbm, age_hbm, gen_hbm, tok_hbm,
          utab_hbm, tstab_hbm, occtab_hbm, bnd_hbm, out_hbm,
          occ_v, ts_v, bnd_v, uid_v, t_v, rate_v, occl_v, age_v, gen_v,
          tok_v, blk_v, out_v, sem):
    wid = lax.axis_index("s") * _NC + lax.axis_index("c")
    base0 = wid * _ROWS_PER_W

    pltpu.sync_copy(occtab_hbm, occ_v.at[pl.ds(0, (_NUM_BUCKETS + 2) * _EMBED_DIM)])
    pltpu.sync_copy(tstab_hbm, ts_v)
    pltpu.sync_copy(bnd_hbm, bnd_v)
    pltpu.sync_copy(uid_hbm.at[pl.ds(base0, _ROWS_PER_W)], uid_v)
    pltpu.sync_copy(t_hbm.at[pl.ds(base0, _ROWS_PER_W)], t_v)
    pltpu.sync_copy(rate_hbm.at[pl.ds(base0, _ROWS_PER_W)], rate_v)
    pltpu.sync_copy(occl_hbm.at[pl.ds(base0, _ROWS_PER_W)], occl_v)
    pltpu.sync_copy(age_hbm.at[pl.ds(base0, _ROWS_PER_W)], age_v)
    pltpu.sync_copy(gen_hbm.at[pl.ds(base0, _ROWS_PER_W)], gen_v)
    pltpu.sync_copy(tok_hbm.at[pl.ds(base0 * _TOK_LEN, _ROWS_PER_W * _TOK_LEN)],
                    tok_v)
    zeros16 = jnp.zeros((16,), jnp.float32)
    occ_v[pl.ds(_ZERO_ROW * _EMBED_DIM, 16)] = zeros16
    occ_v[pl.ds(_ZERO_ROW * _EMBED_DIM + 16, 16)] = zeros16

    iot = lax.iota(jnp.int32, 16)
    iot101 = iot * _OUT_D
    iot20 = iot * _TOK_LEN

    def chunk_body(ci, carry):
        def group_body(g, c2):
            r0 = g * 16
            w0 = ci * _CHUNK + r0
            fi = r0 * _OUT_D + iot101
            us = uid_v[pl.ds(w0, 16)]

            def fire(w):
                for j in range(4):
                    u = us[w * 4 + j]
                    c0 = pl.multiple_of((u >> 7) * 128, 128)
                    pltpu.async_copy(utab_hbm.at[:, pl.ds(c0, 128)],
                                     blk_v.at[(w & 1) * 4 + j], sem)

            def drain_extract(w):
                for j in range(4):
                    pltpu.make_async_copy(utab_hbm.at[:, pl.ds(0, 128)],
                                          blk_v.at[(w & 1) * 4 + j],
                                          sem).wait()
                for j in range(4):
                    u = us[w * 4 + j]
                    col = u & 127
                    rb = (r0 + w * 4 + j) * _OUT_D
                    sj = jnp.full((16,), (w & 1) * 4 + j, jnp.int32)
                    cj = jnp.full((16,), col, jnp.int32)
                    v0 = plsc.load_gather(blk_v, [sj, iot, cj])
                    v1 = plsc.load_gather(blk_v, [sj, iot + 16, cj])
                    out_v[pl.ds(rb, 16)] = v0
                    out_v[pl.ds(rb + 16, 16)] = v1

            fire(0)
            fire(1)

            t = t_v[pl.ds(w0, 16)]
            k0 = jnp.clip((t * float(_NUM_BUCKETS - 1)).astype(jnp.int32) + 1,
                          1, _NUM_BUCKETS)
            b_lo = plsc.load_gather(bnd_v, [k0 - 1])
            b_hi = plsc.load_gather(bnd_v, [k0])
            idx = (k0 - (t < b_lo).astype(jnp.int32)
                   + (t >= b_hi).astype(jnp.int32))
            idx32 = jnp.clip(idx, 0, _NUM_BUCKETS + 1) * _EMBED_DIM

            nt = (t - _NORM_MEAN) / _NORM_STD
            rate = rate_v[pl.ds(w0, 16)]
            occl = occl_v[pl.ds(w0, 16)].astype(jnp.float32)
            age = age_v[pl.ds(w0, 16)]
            gen = gen_v[pl.ds(w0, 16)].astype(jnp.float32)
            plsc.store_scatter(out_v, [fi + 64], nt)
            plsc.store_scatter(out_v, [fi + 65], rate)
            plsc.store_scatter(out_v, [fi + 66], occl)
            plsc.store_scatter(out_v, [fi + 67], age)
            plsc.store_scatter(out_v, [fi + 68], gen)

            tokbase = w0 * _TOK_LEN + iot20
            tok32 = []
            n0 = jnp.zeros((16,), jnp.int32)
            for l in range(_TOK_LEN):
                tk = plsc.load_gather(tok_v, [tokbase + l])
                z = tk == 0
                n0 = n0 + z.astype(jnp.int32)
                tok32.append(jnp.where(z, _ZERO_ROW, tk) * _EMBED_DIM)
            cnt = jnp.maximum(jnp.float32(_TOK_LEN) - n0.astype(jnp.float32), 1.0)
            inv = 1.0 / cnt

            for part in range(4):
                for d in range(part * 8, part * 8 + 8):
                    tvec = plsc.load_gather(ts_v, [idx32 + d])
                    plsc.store_scatter(out_v, [fi + (32 + d)], tvec)
                    accs = [plsc.load_gather(occ_v, [tok32[l] + d])
                            for l in range(4)]
                    for l in range(4, _TOK_LEN):
                        accs[l % 4] = accs[l % 4] + plsc.load_gather(
                            occ_v, [tok32[l] + d])
                    acc = (accs[0] + accs[1]) + (accs[2] + accs[3])
                    plsc.store_scatter(out_v, [fi + (69 + d)], acc * inv)
                drain_extract(part)
                if part < 2:
                    fire(part + 2)
            return c2

        lax.fori_loop(0, _NGROUP, group_body, 0)
        pltpu.sync_copy(out_v,
                        out_hbm.at[pl.ds((base0 + ci * _CHUNK) * _OUT_D,
                                         _CHUNK * _OUT_D)])
        return carry

    lax.fori_loop(0, _NCHUNK, chunk_body, 0)


_sc_call = pl.kernel(
    _body,
    out_type=jax.ShapeDtypeStruct((_BATCH * _OUT_D,), jnp.float32),
    mesh=plsc.VectorSubcoreMesh(core_axis_name="c", subcore_axis_name="s",
                                num_cores=_NC, num_subcores=_NS),
    scratch_types=[
        pltpu.VMEM(((_NUM_BUCKETS + 3) * _EMBED_DIM,), jnp.float32),
        pltpu.VMEM(((_NUM_BUCKETS + 2) * _EMBED_DIM,), jnp.float32),
        pltpu.VMEM((_NUM_BUCKETS + 8,), jnp.float32),
        pltpu.VMEM((_ROWS_PER_W,), jnp.int32),
        pltpu.VMEM((_ROWS_PER_W,), jnp.float32),
        pltpu.VMEM((_ROWS_PER_W,), jnp.float32),
        pltpu.VMEM((_ROWS_PER_W,), jnp.int32),
        pltpu.VMEM((_ROWS_PER_W,), jnp.float32),
        pltpu.VMEM((_ROWS_PER_W,), jnp.int32),
        pltpu.VMEM((_ROWS_PER_W * _TOK_LEN,), jnp.int32),
        pltpu.VMEM((8, _EMBED_DIM, 128), jnp.float32),
        pltpu.VMEM((_CHUNK * _OUT_D,), jnp.float32),
        pltpu.SemaphoreType.DMA,
    ],
    compiler_params=pltpu.CompilerParams(needs_layout_passes=False,
                                         use_tc_tiling_on_sc=True),
)


@jax.jit
def kernel(user_id, timestamp, user_rating, user_occupation_label,
           raw_user_age, user_gender, occ_tokens, user_table, ts_table,
           occ_table):
    boundaries = jnp.linspace(0.0, 1.0, _NUM_BUCKETS).astype(jnp.float32)
    bnd = jnp.concatenate([boundaries, jnp.full((8,), 2.0, jnp.float32)])
    out = _sc_call(user_id, timestamp, user_rating, user_occupation_label,
                   raw_user_age, user_gender, occ_tokens.reshape(-1),
                   user_table.T, ts_table.reshape(-1), occ_table.reshape(-1),
                   bnd)
    return out.reshape(_BATCH, _OUT_D)
